# baseline (speedup 1.0000x reference)
import functools

import jax
import jax.numpy as jnp
from jax import lax
from jax.experimental import pallas as pl
from jax.experimental.pallas import tpu as pltpu
from jax.experimental.pallas import tpu_sc as plsc

D_MODEL = 64
D_PAD = 128
SCALE = 8.0
W = 128
LANES = 16
NBUF = 5
NW = 32
TROW = 129
S_TOT = 200
TB_TOT = 32


def _worker_body(nch, wid, table_hbm, idx_hbm, out_hbm,
                 idx_bufs, row_bufs, out_bufs, idx_sems, g_sems, s_sems):
    iota16 = lax.iota(jnp.int32, LANES)

    def stb(g):
        gg = wid * nch + g
        return gg // TB_TOT, gg % TB_TOT

    def idx_start(g, b):
        s, tb = stb(g)
        pltpu.async_copy(idx_hbm.at[s, pl.ds(tb * W, W)], idx_bufs[b],
                         idx_sems[b])

    def idx_wait(g, b):
        s, tb = stb(g)
        pltpu.make_async_copy(idx_hbm.at[s, pl.ds(tb * W, W)], idx_bufs[b],
                              idx_sems[b]).wait()
        ib = idx_bufs[b]
        for j in range(W // LANES):
            slc = pl.ds(j * LANES, LANES)
            ib.at[slc][...] = ib.at[slc][...] * 2

    def gather_start(b):
        pltpu.async_copy(table_hbm.at[idx_bufs[b]], row_bufs[b], g_sems[b])

    def gather_wait(b):
        pltpu.make_async_copy(table_hbm.at[idx_bufs[b]], row_bufs[b],
                              g_sems[b]).wait()

    def scatter_start(g, b):
        s, tb = stb(g)
        for td in range(8):
            pltpu.async_copy(out_bufs[b].at[pl.ds(td * 8, 8), pl.ds(0, W)],
                             out_hbm.at[s, td, tb], s_sems[b])

    def scatter_wait(g, b):
        s, tb = stb(g)
        for td in range(8):
            pltpu.make_async_copy(
                out_bufs[b].at[pl.ds(td * 8, 8), pl.ds(0, W)],
                out_hbm.at[s, td, tb], s_sems[b]).wait()

    def transform(b):
        rows = row_bufs[b]
        outs = out_bufs[b]

        @plsc.parallel_loop(0, W, unroll=8)
        def _(c):
            cvec = jnp.full((LANES,), 0, jnp.int32) + c
            for q in range(D_MODEL // LANES):
                v = rows.at[c, pl.ds(q * LANES, LANES)][...] * SCALE
                plsc.store_scatter(outs, [iota16 + q * LANES, cvec], v)

    idx_start(0, 0)
    idx_start(1, 1)
    idx_start(2, 2)
    idx_wait(0, 0)
    gather_start(0)
    idx_wait(1, 1)
    gather_start(1)

    def step(g, b):
        @pl.when(g + 3 < nch)
        def _():
            idx_start(g + 3, (b + 3) % NBUF)

        @pl.when(g + 2 < nch)
        def _():
            idx_wait(g + 2, (b + 2) % NBUF)
            gather_start((b + 2) % NBUF)

        gather_wait(b)

        @pl.when(g >= NBUF)
        def _():
            scatter_wait(g - NBUF, b)

        transform(b)
        scatter_start(g, b)

    @pl.loop(0, nch // NBUF)
    def _(i):
        g0 = i * NBUF
        for k in range(NBUF):
            step(g0 + k, k)

    for g in range(nch - NBUF, nch):
        scatter_wait(g, g % NBUF)


def kernel(x, table):
    n_blocks = S_TOT * TB_TOT
    nch = n_blocks // NW
    idx = x.T
    table_p = jnp.pad(table, ((0, 0), (0, D_PAD - D_MODEL)))
    table_v = table_p.reshape(2 * table.shape[0], D_MODEL)
    mesh = plsc.VectorSubcoreMesh(core_axis_name="core",
                                  subcore_axis_name="subcore")

    scratch = (
        [pltpu.VMEM((W,), jnp.int32) for _ in range(NBUF)]
        + [pltpu.VMEM((W, D_MODEL), jnp.float32) for _ in range(NBUF)]
        + [pltpu.VMEM((D_MODEL, TROW), jnp.float32) for _ in range(NBUF)]
        + [pltpu.SemaphoreType.DMA for _ in range(3 * NBUF)]
    )

    @functools.partial(
        pl.kernel,
        out_type=jax.ShapeDtypeStruct((S_TOT, 8, TB_TOT, 8, W), jnp.float32),
        mesh=mesh,
        scratch_types=scratch,
        compiler_params=pltpu.CompilerParams(use_tc_tiling_on_sc=False,
                                             needs_layout_passes=False),
    )
    def run(table_ref, idx_ref, out_ref, *scratch_refs):
        idx_bufs = scratch_refs[0:NBUF]
        row_bufs = scratch_refs[NBUF:2 * NBUF]
        out_bufs = scratch_refs[2 * NBUF:3 * NBUF]
        idx_sems = scratch_refs[3 * NBUF:4 * NBUF]
        g_sems = scratch_refs[4 * NBUF:5 * NBUF]
        s_sems = scratch_refs[5 * NBUF:6 * NBUF]
        wid = lax.axis_index("core") * 16 + lax.axis_index("subcore")
        _worker_body(nch, wid, table_ref, idx_ref, out_ref,
                     idx_bufs, row_bufs, out_bufs, idx_sems, g_sems, s_sems)

    out5 = run(table_v, idx)
    return out5.transpose(2, 4, 0, 1, 3).reshape(x.shape[0], x.shape[1],
                                                 D_MODEL)


# speedup vs baseline: 1.0260x; 1.0260x over previous
import functools

import jax
import jax.numpy as jnp
from jax import lax
from jax.experimental import pallas as pl
from jax.experimental.pallas import tpu as pltpu
from jax.experimental.pallas import tpu_sc as plsc

D_MODEL = 64
D_PAD = 128
SCALE = 8.0
W = 128
LANES = 16
NBUF = 5
NW = 32
TROW = 129
S_TOT = 200
TB_TOT = 32


def _worker_body(nch, wid, table_hbm, idx_hbm, out_hbm,
                 idx_bufs, row_bufs, out_bufs, idx_sems, g_sems, s_sems):
    iota16 = lax.iota(jnp.int32, LANES)

    def stb(g):
        gg = wid * nch + g
        return gg // TB_TOT, gg % TB_TOT

    def idx_start(g, b):
        s, tb = stb(g)
        pltpu.async_copy(idx_hbm.at[s, pl.ds(tb * W, W)], idx_bufs[b],
                         idx_sems[b])

    def idx_wait(g, b):
        s, tb = stb(g)
        pltpu.make_async_copy(idx_hbm.at[s, pl.ds(tb * W, W)], idx_bufs[b],
                              idx_sems[b]).wait()
        ib = idx_bufs[b]
        for j in range(W // LANES):
            slc = pl.ds(j * LANES, LANES)
            ib.at[slc][...] = ib.at[slc][...] * 2

    def gather_start(b):
        pltpu.async_copy(table_hbm.at[idx_bufs[b]], row_bufs[b], g_sems[b])

    def gather_wait(b):
        pltpu.make_async_copy(table_hbm.at[idx_bufs[b]], row_bufs[b],
                              g_sems[b]).wait()

    def scatter_start(g, b):
        s, tb = stb(g)
        for td in range(8):
            pltpu.async_copy(out_bufs[b].at[pl.ds(td * 8, 8), pl.ds(0, W)],
                             out_hbm.at[s, td, tb], s_sems[b])

    def scatter_wait(g, b):
        s, tb = stb(g)
        for td in range(8):
            pltpu.make_async_copy(
                out_bufs[b].at[pl.ds(td * 8, 8), pl.ds(0, W)],
                out_hbm.at[s, td, tb], s_sems[b]).wait()

    def transform(b):
        rows = row_bufs[b]
        outs = out_bufs[b]

        @plsc.parallel_loop(0, W, unroll=4)
        def _(c):
            cvec = jnp.full((LANES,), 0, jnp.int32) + c
            for q in range(D_MODEL // LANES):
                v = rows.at[c, pl.ds(q * LANES, LANES)][...] * SCALE
                plsc.store_scatter(outs, [iota16 + q * LANES, cvec], v)

    idx_start(0, 0)
    idx_start(1, 1)
    idx_start(2, 2)
    idx_start(3, 3)
    idx_wait(0, 0)
    gather_start(0)
    idx_wait(1, 1)
    gather_start(1)
    idx_wait(2, 2)
    gather_start(2)

    def step(g, b):
        @pl.when(g + 4 < nch)
        def _():
            idx_start(g + 4, (b + 4) % NBUF)

        @pl.when(g + 3 < nch)
        def _():
            idx_wait(g + 3, (b + 3) % NBUF)
            gather_start((b + 3) % NBUF)

        gather_wait(b)

        @pl.when(g >= NBUF)
        def _():
            scatter_wait(g - NBUF, b)

        transform(b)
        scatter_start(g, b)

    @pl.loop(0, nch // NBUF)
    def _(i):
        g0 = i * NBUF
        for k in range(NBUF):
            step(g0 + k, k)

    for g in range(nch - NBUF, nch):
        scatter_wait(g, g % NBUF)


def kernel(x, table):
    n_blocks = S_TOT * TB_TOT
    nch = n_blocks // NW
    idx = x.T
    table_p = jnp.pad(table, ((0, 0), (0, D_PAD - D_MODEL)))
    table_v = table_p.reshape(2 * table.shape[0], D_MODEL)
    mesh = plsc.VectorSubcoreMesh(core_axis_name="core",
                                  subcore_axis_name="subcore")

    scratch = (
        [pltpu.VMEM((W,), jnp.int32) for _ in range(NBUF)]
        + [pltpu.VMEM((W, D_MODEL), jnp.float32) for _ in range(NBUF)]
        + [pltpu.VMEM((D_MODEL, TROW), jnp.float32) for _ in range(NBUF)]
        + [pltpu.SemaphoreType.DMA for _ in range(3 * NBUF)]
    )

    @functools.partial(
        pl.kernel,
        out_type=jax.ShapeDtypeStruct((S_TOT, 8, TB_TOT, 8, W), jnp.float32),
        mesh=mesh,
        scratch_types=scratch,
        compiler_params=pltpu.CompilerParams(use_tc_tiling_on_sc=False,
                                             needs_layout_passes=False),
    )
    def run(table_ref, idx_ref, out_ref, *scratch_refs):
        idx_bufs = scratch_refs[0:NBUF]
        row_bufs = scratch_refs[NBUF:2 * NBUF]
        out_bufs = scratch_refs[2 * NBUF:3 * NBUF]
        idx_sems = scratch_refs[3 * NBUF:4 * NBUF]
        g_sems = scratch_refs[4 * NBUF:5 * NBUF]
        s_sems = scratch_refs[5 * NBUF:6 * NBUF]
        wid = lax.axis_index("core") * 16 + lax.axis_index("subcore")
        _worker_body(nch, wid, table_ref, idx_ref, out_ref,
                     idx_bufs, row_bufs, out_bufs, idx_sems, g_sems, s_sems)

    out5 = run(table_v, idx)
    return out5.transpose(2, 4, 0, 1, 3).reshape(x.shape[0], x.shape[1],
                                                 D_MODEL)
